# Initial kernel scaffold; baseline (speedup 1.0000x reference)
import jax
import jax.numpy as jnp
from jax.experimental import pallas as pl


def kernel(x, edge_index, edge_attr, return_attention_weights, W, b):
    raise NotImplementedError("write your pallas kernel here")


# speedup vs baseline: 11.1358x; 11.1358x over previous
import functools

import jax
import jax.numpy as jnp
from jax import lax
from jax.experimental import pallas as pl
from jax.experimental.pallas import tpu as pltpu
from jax.experimental.pallas import tpu_sc as plsc

N_NODES = 10000
N_EDGES = 320000
D = 128

NCORES = 2
NSUB = 16
NW = NCORES * NSUB

NP = 10240
CH = 128
NCH = 80
EP = NW * NCH * CH
DUMMY = N_NODES

ROWS_PER_TILE = NP // NSUB

_mesh = plsc.VectorSubcoreMesh(core_axis_name="c", subcore_axis_name="s")


@functools.partial(
    pl.kernel,
    out_type=jax.ShapeDtypeStruct((NCORES, NP), jnp.float32),
    mesh=_mesh,
    scratch_types=[
        pltpu.VMEM((NCH, CH), jnp.int32),
        pltpu.VMEM((CH,), jnp.float32),
        pltpu.VMEM_SHARED((NP,), jnp.float32),
    ],
)
def _sc_deg(col_hbm, zeros1_hbm, ones_hbm, out_hbm, colv, onesv, deg_sh):
    cid = lax.axis_index("c")
    sid = lax.axis_index("s")
    wid = cid * NSUB + sid
    base = sid * ROWS_PER_TILE
    pltpu.sync_copy(zeros1_hbm.at[pl.ds(base, ROWS_PER_TILE)],
                    deg_sh.at[pl.ds(base, ROWS_PER_TILE)])
    pltpu.sync_copy(col_hbm.at[wid], colv)
    pltpu.sync_copy(ones_hbm, onesv)
    plsc.subcore_barrier()

    @pl.loop(0, NCH)
    def _(j):
        pltpu.sync_copy(onesv, deg_sh.at[colv.at[j]], add=True)

    plsc.subcore_barrier()
    pltpu.sync_copy(deg_sh.at[pl.ds(base, ROWS_PER_TILE)],
                    out_hbm.at[cid, pl.ds(base, ROWS_PER_TILE)])


@functools.partial(
    pl.kernel,
    out_type=jax.ShapeDtypeStruct((NCORES, NP, D), jnp.float32),
    mesh=_mesh,
    scratch_types=[
        pltpu.VMEM((NCH, CH), jnp.int32),
        pltpu.VMEM((NCH, CH), jnp.int32),
        pltpu.VMEM((CH, D), jnp.float32),
        pltpu.VMEM_SHARED((NP, D), jnp.float32),
    ],
)
def _sc_main(g_hbm, row_hbm, col_hbm, zeros2_hbm, out_hbm,
             rowv, colv, rowsbuf, acc_sh):
    cid = lax.axis_index("c")
    sid = lax.axis_index("s")
    wid = cid * NSUB + sid
    base = sid * ROWS_PER_TILE
    pltpu.sync_copy(zeros2_hbm.at[pl.ds(base, ROWS_PER_TILE)],
                    acc_sh.at[pl.ds(base, ROWS_PER_TILE)])
    pltpu.sync_copy(row_hbm.at[wid], rowv)
    pltpu.sync_copy(col_hbm.at[wid], colv)
    plsc.subcore_barrier()

    @pl.loop(0, NCH)
    def _(j):
        pltpu.sync_copy(g_hbm.at[rowv.at[j]], rowsbuf)
        pltpu.sync_copy(rowsbuf, acc_sh.at[colv.at[j]], add=True)

    plsc.subcore_barrier()
    pltpu.sync_copy(acc_sh.at[pl.ds(base, ROWS_PER_TILE)],
                    out_hbm.at[cid, pl.ds(base, ROWS_PER_TILE)])


BR = 1000


def _dis_from(degT_blk):
    d = degT_blk[:, 0:1] + degT_blk[:, 1:2]
    return jnp.where(d > 0, lax.rsqrt(jnp.maximum(d, 1e-12)), 0.0)


def _scale_body(x_ref, w_ref, degT_ref, g_ref):
    h = jnp.dot(x_ref[...], w_ref[...], preferred_element_type=jnp.float32)
    g_ref[...] = h * _dis_from(degT_ref[...])


_tc_scale = pl.pallas_call(
    _scale_body,
    grid=(N_NODES // BR,),
    in_specs=[
        pl.BlockSpec((BR, D), lambda i: (i, 0)),
        pl.BlockSpec((D, D), lambda i: (0, 0)),
        pl.BlockSpec((BR, 2), lambda i: (i, 0)),
    ],
    out_specs=pl.BlockSpec((BR, D), lambda i: (i, 0)),
    out_shape=jax.ShapeDtypeStruct((N_NODES, D), jnp.float32),
)


def _final_body(acc_ref, degT_ref, b_ref, out_ref):
    a = acc_ref[0] + acc_ref[1]
    out_ref[...] = a * _dis_from(degT_ref[...]) + b_ref[...]


_tc_final = pl.pallas_call(
    _final_body,
    grid=(N_NODES // BR,),
    in_specs=[
        pl.BlockSpec((NCORES, BR, D), lambda i: (0, i, 0)),
        pl.BlockSpec((BR, 2), lambda i: (i, 0)),
        pl.BlockSpec((1, D), lambda i: (0, 0)),
    ],
    out_specs=pl.BlockSpec((BR, D), lambda i: (i, 0)),
    out_shape=jax.ShapeDtypeStruct((N_NODES, D), jnp.float32),
)


def kernel(x, edge_index, edge_attr, return_attention_weights, W, b):
    del edge_attr, return_attention_weights
    row = edge_index[0]
    col = edge_index[1]
    npad = EP - N_EDGES
    row_p = jnp.concatenate([row, jnp.zeros((npad,), jnp.int32)])
    col_p = jnp.concatenate([col, jnp.full((npad,), DUMMY, jnp.int32)])
    row3 = row_p.reshape(NW, NCH, CH)
    col3 = col_p.reshape(NW, NCH, CH)

    zeros1 = jnp.zeros((NP,), jnp.float32)
    ones_ch = jnp.ones((CH,), jnp.float32)
    zeros2 = jnp.zeros((NP, D), jnp.float32)

    deg2 = _sc_deg(col3, zeros1, ones_ch)
    degT = deg2.T
    g = _tc_scale(x, W, degT)
    acc = _sc_main(g, row3, col3, zeros2)
    out = _tc_final(acc, degT, b.reshape(1, D))
    return out


# speedup vs baseline: 11.8880x; 1.0675x over previous
import functools

import jax
import jax.numpy as jnp
from jax import lax
from jax.experimental import pallas as pl
from jax.experimental.pallas import tpu as pltpu
from jax.experimental.pallas import tpu_sc as plsc

N_NODES = 10000
N_EDGES = 320000
D = 128

NCORES = 2
NSUB = 16
NW = NCORES * NSUB

NP = 10240
CH = 128
NCH = 80
EP = NW * NCH * CH
DUMMY = N_NODES

ROWS_PER_TILE = NP // NSUB

_mesh = plsc.VectorSubcoreMesh(core_axis_name="c", subcore_axis_name="s")


@functools.partial(
    pl.kernel,
    out_type=jax.ShapeDtypeStruct((NCORES, NP), jnp.float32),
    mesh=_mesh,
    scratch_types=[
        pltpu.VMEM((NCH, CH), jnp.int32),
        pltpu.VMEM((CH,), jnp.float32),
        pltpu.VMEM_SHARED((NP,), jnp.float32),
    ],
)
def _sc_deg(col_hbm, zeros1_hbm, ones_hbm, out_hbm, colv, onesv, deg_sh):
    cid = lax.axis_index("c")
    sid = lax.axis_index("s")
    wid = cid * NSUB + sid
    base = sid * ROWS_PER_TILE
    pltpu.sync_copy(zeros1_hbm.at[pl.ds(base, ROWS_PER_TILE)],
                    deg_sh.at[pl.ds(base, ROWS_PER_TILE)])
    pltpu.sync_copy(col_hbm.at[wid], colv)
    pltpu.sync_copy(ones_hbm, onesv)
    plsc.subcore_barrier()

    @pl.loop(0, NCH)
    def _(j):
        pltpu.sync_copy(onesv, deg_sh.at[colv.at[j]], add=True)

    plsc.subcore_barrier()
    pltpu.sync_copy(deg_sh.at[pl.ds(base, ROWS_PER_TILE)],
                    out_hbm.at[cid, pl.ds(base, ROWS_PER_TILE)])


NPH = 2
PCH = NCH // NPH


@functools.partial(
    pl.kernel,
    out_type=jax.ShapeDtypeStruct((NCORES, NP, D), jnp.float32),
    mesh=_mesh,
    scratch_types=[
        pltpu.VMEM((PCH, CH), jnp.int32),
        pltpu.VMEM((PCH, CH), jnp.int32),
        pltpu.VMEM((CH, D), jnp.float32),
        pltpu.VMEM((CH, D), jnp.float32),
        pltpu.SemaphoreType.DMA,
        pltpu.SemaphoreType.DMA,
        pltpu.SemaphoreType.DMA,
        pltpu.SemaphoreType.DMA,
        pltpu.VMEM_SHARED((NP, D), jnp.float32),
    ],
)
def _sc_main(g_hbm, row_hbm, col_hbm, zeros2_hbm, out_hbm,
             rowv, colv, buf0, buf1, gsem0, gsem1, ssem0, ssem1, acc_sh):
    cid = lax.axis_index("c")
    sid = lax.axis_index("s")
    wid = cid * NSUB + sid
    base = sid * ROWS_PER_TILE
    pltpu.sync_copy(zeros2_hbm.at[pl.ds(base, ROWS_PER_TILE)],
                    acc_sh.at[pl.ds(base, ROWS_PER_TILE)])
    plsc.subcore_barrier()

    bufs = (buf0, buf1)
    gsems = (gsem0, gsem1)
    ssems = (ssem0, ssem1)

    for ph in range(NPH):
        pltpu.sync_copy(row_hbm.at[wid, pl.ds(ph * PCH, PCH)], rowv)
        pltpu.sync_copy(col_hbm.at[wid, pl.ds(ph * PCH, PCH)], colv)

        @pl.loop(0, PCH, step=2)
        def _(go):
            for p in range(2):
                grp = go + p
                buf, gsem, ssem = bufs[p], gsems[p], ssems[p]

                @pl.when(grp >= 2)
                def _():
                    pltpu.make_async_copy(
                        buf, acc_sh.at[colv.at[grp - 2]], ssem).wait()

                pltpu.async_copy(g_hbm.at[rowv.at[grp]], buf, gsem)
                pltpu.make_async_copy(g_hbm.at[rowv.at[grp]], buf, gsem).wait()

                pltpu.async_copy(buf, acc_sh.at[colv.at[grp]], ssem, add=True)

        for p in range(2):
            pltpu.make_async_copy(bufs[p], acc_sh.at[colv.at[PCH - 2 + p]],
                                  ssems[p]).wait()

    plsc.subcore_barrier()
    pltpu.sync_copy(acc_sh.at[pl.ds(base, ROWS_PER_TILE)],
                    out_hbm.at[cid, pl.ds(base, ROWS_PER_TILE)])


BR = 1000


def _dis_from(degT_blk):
    d = degT_blk[:, 0:1] + degT_blk[:, 1:2]
    return jnp.where(d > 0, lax.rsqrt(jnp.maximum(d, 1e-12)), 0.0)


def _scale_body(x_ref, w_ref, degT_ref, g_ref):
    h = jnp.dot(x_ref[...], w_ref[...], preferred_element_type=jnp.float32)
    g_ref[...] = h * _dis_from(degT_ref[...])


_tc_scale = pl.pallas_call(
    _scale_body,
    grid=(N_NODES // BR,),
    in_specs=[
        pl.BlockSpec((BR, D), lambda i: (i, 0)),
        pl.BlockSpec((D, D), lambda i: (0, 0)),
        pl.BlockSpec((BR, 2), lambda i: (i, 0)),
    ],
    out_specs=pl.BlockSpec((BR, D), lambda i: (i, 0)),
    out_shape=jax.ShapeDtypeStruct((N_NODES, D), jnp.float32),
)


def _final_body(acc_ref, degT_ref, b_ref, out_ref):
    a = acc_ref[0] + acc_ref[1]
    out_ref[...] = a * _dis_from(degT_ref[...]) + b_ref[...]


_tc_final = pl.pallas_call(
    _final_body,
    grid=(N_NODES // BR,),
    in_specs=[
        pl.BlockSpec((NCORES, BR, D), lambda i: (0, i, 0)),
        pl.BlockSpec((BR, 2), lambda i: (i, 0)),
        pl.BlockSpec((1, D), lambda i: (0, 0)),
    ],
    out_specs=pl.BlockSpec((BR, D), lambda i: (i, 0)),
    out_shape=jax.ShapeDtypeStruct((N_NODES, D), jnp.float32),
)


def kernel(x, edge_index, edge_attr, return_attention_weights, W, b):
    del edge_attr, return_attention_weights
    row = edge_index[0]
    col = edge_index[1]
    npad = EP - N_EDGES
    row_p = jnp.concatenate([row, jnp.zeros((npad,), jnp.int32)])
    col_p = jnp.concatenate([col, jnp.full((npad,), DUMMY, jnp.int32)])
    row3 = row_p.reshape(NW, NCH, CH)
    col3 = col_p.reshape(NW, NCH, CH)

    zeros1 = jnp.zeros((NP,), jnp.float32)
    ones_ch = jnp.ones((CH,), jnp.float32)
    zeros2 = jnp.zeros((NP, D), jnp.float32)

    deg2 = _sc_deg(col3, zeros1, ones_ch)
    degT = deg2.T
    g = _tc_scale(x, W, degT)
    acc = _sc_main(g, row3, col3, zeros2)
    out = _tc_final(acc, degT, b.reshape(1, D))
    return out


# speedup vs baseline: 12.6187x; 1.0615x over previous
import functools

import jax
import jax.numpy as jnp
from jax import lax
from jax.experimental import pallas as pl
from jax.experimental.pallas import tpu as pltpu
from jax.experimental.pallas import tpu_sc as plsc

N_NODES = 10000
N_EDGES = 320000
D = 128

NCORES = 2
NSUB = 16
NW = NCORES * NSUB

NP = 10240
DCH = 128
DNCH = 80
CH = 64
NCH = 160
EP = NW * NCH * CH
DUMMY = N_NODES

ROWS_PER_TILE = NP // NSUB

_mesh = plsc.VectorSubcoreMesh(core_axis_name="c", subcore_axis_name="s")


@functools.partial(
    pl.kernel,
    out_type=jax.ShapeDtypeStruct((NCORES, NP), jnp.float32),
    mesh=_mesh,
    scratch_types=[
        pltpu.VMEM((DNCH, DCH), jnp.int32),
        pltpu.VMEM((DCH,), jnp.float32),
        pltpu.VMEM_SHARED((NP,), jnp.float32),
    ],
)
def _sc_deg(col_hbm, zeros1_hbm, ones_hbm, out_hbm, colv, onesv, deg_sh):
    cid = lax.axis_index("c")
    sid = lax.axis_index("s")
    wid = cid * NSUB + sid
    base = sid * ROWS_PER_TILE
    pltpu.sync_copy(zeros1_hbm.at[pl.ds(base, ROWS_PER_TILE)],
                    deg_sh.at[pl.ds(base, ROWS_PER_TILE)])
    pltpu.sync_copy(col_hbm.at[wid], colv)
    pltpu.sync_copy(ones_hbm, onesv)
    plsc.subcore_barrier()

    @pl.loop(0, DNCH)
    def _(j):
        pltpu.sync_copy(onesv, deg_sh.at[colv.at[j]], add=True)

    plsc.subcore_barrier()
    pltpu.sync_copy(deg_sh.at[pl.ds(base, ROWS_PER_TILE)],
                    out_hbm.at[cid, pl.ds(base, ROWS_PER_TILE)])


NPH = 4
PCH = NCH // NPH


@functools.partial(
    pl.kernel,
    out_type=jax.ShapeDtypeStruct((NCORES, NP, D), jnp.float32),
    mesh=_mesh,
    scratch_types=[
        pltpu.VMEM((PCH, CH), jnp.int32),
        pltpu.VMEM((PCH, CH), jnp.int32),
        [pltpu.VMEM((CH, D), jnp.float32)] * 4,
        [pltpu.SemaphoreType.DMA] * 4,
        [pltpu.SemaphoreType.DMA] * 4,
        pltpu.VMEM_SHARED((NP, D), jnp.float32),
    ],
)
def _sc_main(g_hbm, row_hbm, col_hbm, zeros2_hbm, out_hbm,
             rowv, colv, bufs, gsems, ssems, acc_sh):
    cid = lax.axis_index("c")
    sid = lax.axis_index("s")
    wid = cid * NSUB + sid
    base = sid * ROWS_PER_TILE
    pltpu.sync_copy(zeros2_hbm.at[pl.ds(base, ROWS_PER_TILE)],
                    acc_sh.at[pl.ds(base, ROWS_PER_TILE)])
    plsc.subcore_barrier()

    for ph in range(NPH):
        pltpu.sync_copy(row_hbm.at[wid, pl.ds(ph * PCH, PCH)], rowv)
        pltpu.sync_copy(col_hbm.at[wid, pl.ds(ph * PCH, PCH)], colv)

        pltpu.async_copy(g_hbm.at[rowv.at[0]], bufs[0], gsems[0])
        pltpu.async_copy(g_hbm.at[rowv.at[1]], bufs[1], gsems[1])

        @pl.loop(0, PCH, step=4)
        def _(go):
            for ds in range(4):
                s = ds
                s2 = (ds + 2) % 4
                grp = go + ds

                pltpu.make_async_copy(g_hbm.at[rowv.at[grp]],
                                      bufs[s], gsems[s]).wait()
                pltpu.async_copy(bufs[s], acc_sh.at[colv.at[grp]],
                                 ssems[s], add=True)

                @pl.when(grp + 2 < PCH)
                def _():
                    @pl.when(grp >= 2)
                    def _():
                        pltpu.make_async_copy(
                            bufs[s2], acc_sh.at[colv.at[grp - 2]],
                            ssems[s2]).wait()
                    pltpu.async_copy(g_hbm.at[rowv.at[grp + 2]],
                                     bufs[s2], gsems[s2])

        for i in range(4):
            pltpu.make_async_copy(bufs[i], acc_sh.at[colv.at[PCH - 4 + i]],
                                  ssems[i]).wait()

    plsc.subcore_barrier()
    pltpu.sync_copy(acc_sh.at[pl.ds(base, ROWS_PER_TILE)],
                    out_hbm.at[cid, pl.ds(base, ROWS_PER_TILE)])


BR = 1000


def _dis_from(degT_blk):
    d = degT_blk[:, 0:1] + degT_blk[:, 1:2]
    return jnp.where(d > 0, lax.rsqrt(jnp.maximum(d, 1e-12)), 0.0)


def _scale_body(x_ref, w_ref, degT_ref, g_ref):
    h = jnp.dot(x_ref[...], w_ref[...], preferred_element_type=jnp.float32)
    g_ref[...] = h * _dis_from(degT_ref[...])


_tc_scale = pl.pallas_call(
    _scale_body,
    grid=(N_NODES // BR,),
    in_specs=[
        pl.BlockSpec((BR, D), lambda i: (i, 0)),
        pl.BlockSpec((D, D), lambda i: (0, 0)),
        pl.BlockSpec((BR, 2), lambda i: (i, 0)),
    ],
    out_specs=pl.BlockSpec((BR, D), lambda i: (i, 0)),
    out_shape=jax.ShapeDtypeStruct((N_NODES, D), jnp.float32),
)


def _final_body(acc_ref, degT_ref, b_ref, out_ref):
    a = acc_ref[0] + acc_ref[1]
    out_ref[...] = a * _dis_from(degT_ref[...]) + b_ref[...]


_tc_final = pl.pallas_call(
    _final_body,
    grid=(N_NODES // BR,),
    in_specs=[
        pl.BlockSpec((NCORES, BR, D), lambda i: (0, i, 0)),
        pl.BlockSpec((BR, 2), lambda i: (i, 0)),
        pl.BlockSpec((1, D), lambda i: (0, 0)),
    ],
    out_specs=pl.BlockSpec((BR, D), lambda i: (i, 0)),
    out_shape=jax.ShapeDtypeStruct((N_NODES, D), jnp.float32),
)


def kernel(x, edge_index, edge_attr, return_attention_weights, W, b):
    del edge_attr, return_attention_weights
    row = edge_index[0]
    col = edge_index[1]
    npad = EP - N_EDGES
    row_p = jnp.concatenate([row, jnp.zeros((npad,), jnp.int32)])
    col_p = jnp.concatenate([col, jnp.full((npad,), DUMMY, jnp.int32)])
    row3 = row_p.reshape(NW, NCH, CH)
    col3 = col_p.reshape(NW, NCH, CH)
    col3d = col_p.reshape(NW, DNCH, DCH)

    zeros1 = jnp.zeros((NP,), jnp.float32)
    ones_ch = jnp.ones((DCH,), jnp.float32)
    zeros2 = jnp.zeros((NP, D), jnp.float32)

    deg2 = _sc_deg(col3d, zeros1, ones_ch)
    degT = deg2.T
    g = _tc_scale(x, W, degT)
    acc = _sc_main(g, row3, col3, zeros2)
    out = _tc_final(acc, degT, b.reshape(1, D))
    return out
